# baseline (speedup 1.0000x reference)
import jax
import jax.numpy as jnp
import numpy as np
from jax.experimental import pallas as pl

_N = 1024
_C = 8
_GAMMA = 1.0
_PER = 0.001
_SIG1 = float(1.0 / (1.0 + np.exp(-1.0)))


def _loss_body(preds_ref, lab_ref, mask_ref, a_ref, aself_ref, diag8_ref,
               out_ref):
    preds = preds_ref[...]
    lab8 = lab_ref[...]
    mask8 = mask_ref[...]
    diag8 = diag8_ref[...]

    mx = jnp.max(preds, axis=1, keepdims=True)
    lse = jnp.log(jnp.sum(jnp.exp(preds - mx), axis=1, keepdims=True)) + mx
    logp = preds - lse
    cls_iota = jax.lax.broadcasted_iota(jnp.int32, (_N, _C), 1)
    lab_oh = (cls_iota == lab8).astype(jnp.float32)
    ce = -jnp.sum(logp * lab_oh) * (1.0 / _N)

    m_cls = lab_oh * mask8
    ncnt = jnp.sum(m_cls, axis=0, keepdims=True)
    m_bf = m_cls.astype(jnp.bfloat16)

    eg = jnp.exp(_GAMMA * preds)
    u_bf = (m_cls / eg).astype(jnp.bfloat16)

    a_bf = a_ref[...].astype(jnp.bfloat16)
    aself_bf = aself_ref[...].astype(jnp.bfloat16)
    inter = jax.lax.dot_general(a_bf, a_bf, (((1,), (0,)), ((), ())),
                                preferred_element_type=jnp.float32
                                ).astype(jnp.bfloat16)
    cross = jax.lax.dot_general(a_bf, aself_bf, (((1,), (0,)), ((), ())),
                                preferred_element_type=jnp.float32
                                ).astype(jnp.bfloat16)
    wt = jax.lax.dot_general(a_bf, m_bf, (((1,), (0,)), ((), ())),
                             preferred_element_type=jnp.float32)
    ones_row = jnp.ones((1, _N), dtype=jnp.bfloat16)
    deg_row = jax.lax.dot_general(ones_row, a_bf, (((1,), (0,)), ((), ())),
                                  preferred_element_type=jnp.float32
                                  ).astype(jnp.bfloat16)

    s = jnp.bfloat16(_SIG1)
    one = jnp.bfloat16(1.0)
    base_row = one + s * deg_row
    num = base_row - s * cross
    den = one + s * inter
    v_bf = one / (one + jnp.exp(num / den))

    t = jax.lax.dot_general(v_bf, u_bf, (((1,), (0,)), ((), ())),
                            preferred_element_type=jnp.float32)
    p_t = jax.lax.dot_general(m_cls, t * eg, (((0,), (0,)), ((), ())),
                              preferred_element_type=jnp.float32)

    zt = ncnt - wt - m_cls * (1.0 - diag8)
    g_inter = jax.lax.dot_general(wt, wt, (((0,), (0,)), ((), ())),
                                  preferred_element_type=jnp.float32)
    g_sub_t = jax.lax.dot_general(zt, wt, (((0,), (0,)), ((), ())),
                                  preferred_element_type=jnp.float32)

    denom = jnp.reshape(ncnt, (_C, 1)) * ncnt
    recip = jnp.where(denom > 0.0, 1.0 / jnp.where(denom > 0.0, denom, 1.0), 0.0)
    ii = jax.lax.broadcasted_iota(jnp.int32, (_C, _C), 0)
    jj = jax.lax.broadcasted_iota(jnp.int32, (_C, _C), 1)
    keep = jnp.logical_and(jnp.logical_and(g_sub_t > 0.0, g_inter > 0.0),
                           ii != jj)
    pair_loss = jnp.sum(jnp.where(keep, p_t * recip, 0.0))

    out_ref[...] = jnp.reshape(ce + _PER * pair_loss, (1, 1))


def kernel(preds, labels, mask, w_values_dict, adj_matrix):
    del w_values_dict
    adj_b = adj_matrix.astype(bool)
    aself_b = jnp.logical_or(adj_b, jnp.eye(_N, dtype=bool))
    diag8 = jnp.broadcast_to(jnp.diagonal(adj_b).astype(jnp.float32)[:, None],
                             (_N, _C))
    lab8 = jnp.broadcast_to(labels.astype(jnp.int32)[:, None], (_N, _C))
    mask8 = jnp.broadcast_to(mask.astype(jnp.float32)[:, None], (_N, _C))
    out = pl.pallas_call(
        _loss_body,
        out_shape=jax.ShapeDtypeStruct((1, 1), jnp.float32),
    )(preds.astype(jnp.float32), lab8, mask8, adj_b, aself_b, diag8)
    return out[0, 0]


# speedup vs baseline: 2.7127x; 2.7127x over previous
import jax
import jax.numpy as jnp
import numpy as np
from jax.experimental import pallas as pl

_N = 1024
_C = 8
_GAMMA = 1.0
_PER = 0.001
_SIG1 = float(1.0 / (1.0 + np.exp(-1.0)))


def _loss_body(preds_ref, lab_ref, mask_ref, a_ref, diagrow_ref, diag8_ref,
               out_ref):
    preds = preds_ref[...]
    lab8 = lab_ref[...]
    mask8 = mask_ref[...]
    diag_row = diagrow_ref[...]
    diag8 = diag8_ref[...]

    mx = jnp.max(preds, axis=1, keepdims=True)
    lse = jnp.log(jnp.sum(jnp.exp(preds - mx), axis=1, keepdims=True)) + mx
    logp = preds - lse
    cls_iota = jax.lax.broadcasted_iota(jnp.int32, (_N, _C), 1)
    lab_oh = (cls_iota == lab8).astype(jnp.float32)
    ce = -jnp.sum(logp * lab_oh) * (1.0 / _N)

    m_cls = lab_oh * mask8
    ncnt = jnp.sum(m_cls, axis=0, keepdims=True)
    m_bf = m_cls.astype(jnp.bfloat16)

    eg = jnp.exp(_GAMMA * preds)
    u_bf = (m_cls / eg).astype(jnp.bfloat16)

    a_bf = a_ref[...]
    inter = jax.lax.dot_general(a_bf, a_bf, (((1,), (0,)), ((), ())),
                                preferred_element_type=jnp.float32
                                ).astype(jnp.bfloat16)
    wt = jax.lax.dot_general(a_bf, m_bf, (((1,), (0,)), ((), ())),
                             preferred_element_type=jnp.float32)
    ones_row = jnp.ones((1, _N), dtype=jnp.bfloat16)
    deg_row = jax.lax.dot_general(ones_row, a_bf, (((1,), (0,)), ((), ())),
                                  preferred_element_type=jnp.float32
                                  ).astype(jnp.bfloat16)

    s = jnp.bfloat16(_SIG1)
    one = jnp.bfloat16(1.0)
    base1_row = jnp.bfloat16(2.0) + s * deg_row
    coef_row = s * (one - diag_row.astype(jnp.bfloat16))
    den = one + s * inter
    num = base1_row - den - a_bf * coef_row
    v_bf = one / (one + jnp.exp(num / den))

    t = jax.lax.dot_general(v_bf, u_bf, (((1,), (0,)), ((), ())),
                            preferred_element_type=jnp.float32)
    p_t = jax.lax.dot_general(m_cls, t * eg, (((0,), (0,)), ((), ())),
                              preferred_element_type=jnp.float32)

    zt = ncnt - wt - m_cls * (1.0 - diag8)
    g_inter = jax.lax.dot_general(wt, wt, (((0,), (0,)), ((), ())),
                                  preferred_element_type=jnp.float32)
    g_sub_t = jax.lax.dot_general(zt, wt, (((0,), (0,)), ((), ())),
                                  preferred_element_type=jnp.float32)

    denom = jnp.reshape(ncnt, (_C, 1)) * ncnt
    recip = jnp.where(denom > 0.0, 1.0 / jnp.where(denom > 0.0, denom, 1.0), 0.0)
    ii = jax.lax.broadcasted_iota(jnp.int32, (_C, _C), 0)
    jj = jax.lax.broadcasted_iota(jnp.int32, (_C, _C), 1)
    keep = jnp.logical_and(jnp.logical_and(g_sub_t > 0.0, g_inter > 0.0),
                           ii != jj)
    pair_loss = jnp.sum(jnp.where(keep, p_t * recip, 0.0))

    out_ref[...] = jnp.reshape(ce + _PER * pair_loss, (1, 1))


def kernel(preds, labels, mask, w_values_dict, adj_matrix):
    del w_values_dict
    adj_b = adj_matrix.astype(bool)
    eye = jnp.eye(_N, dtype=bool)
    diagv = jnp.sum(jnp.logical_and(adj_b, eye), axis=1).astype(jnp.float32)
    diag_row = diagv.reshape(1, _N)
    diag8 = jnp.broadcast_to(diagv[:, None], (_N, _C))
    lab8 = jnp.broadcast_to(labels.astype(jnp.int32)[:, None], (_N, _C))
    mask8 = jnp.broadcast_to(mask.astype(jnp.float32)[:, None], (_N, _C))
    out = pl.pallas_call(
        _loss_body,
        out_shape=jax.ShapeDtypeStruct((1, 1), jnp.float32),
    )(preds.astype(jnp.float32), lab8, mask8, adj_b.astype(jnp.bfloat16),
      diag_row, diag8)
    return out[0, 0]
